# Initial kernel scaffold; baseline (speedup 1.0000x reference)
import jax
import jax.numpy as jnp
from jax.experimental import pallas as pl


def kernel(feats0, feats1, feats2, feats3, feats4, rpn_conv_w, rpn_conv_b, rpn_cls_w, rpn_cls_b, rpn_reg_w, rpn_reg_b, fc1_w, fc1_b, fc2_w, fc2_b, cls_w, cls_b, reg_w, reg_b):
    raise NotImplementedError("write your pallas kernel here")


# speedup vs baseline: 1.0892x; 1.0892x over previous
import functools

import numpy as np
import jax
import jax.numpy as jnp
from jax import lax
from jax.experimental import pallas as pl
from jax.experimental.pallas import tpu as pltpu

H_IMG, W_IMG = 832.0, 640.0
STRIDES = (4, 8, 16, 32, 64)
SIZES = (64.0, 128.0, 256.0, 512.0, 1024.0)
RATIOS = (0.04, 0.05, 0.08, 0.15)
A = len(SIZES) * len(RATIOS)
PRE_NMS = 100
POST_NMS = 80
RPN_NMS_TH = 0.7
BOX_NMS_TH = 0.5
SCORE_TH = 0.05
POOL = 7
SAMPLING = 2
CANONICAL = 224.0
BBOX_W = (10.0, 10.0, 5.0, 5.0)
LOG_CLAMP = float(np.log(1000.0 / 16.0))

C = 256


def _base_anchors_np():
    scales = np.array(SIZES, np.float32)
    ratios = np.array(RATIOS, np.float32)
    h_r = np.sqrt(ratios)
    w_r = 1.0 / h_r
    ws = (w_r[:, None] * scales[None, :]).reshape(-1)
    hs = (h_r[:, None] * scales[None, :]).reshape(-1)
    return np.round(np.stack([-ws, -hs, ws, hs], axis=1) / 2.0)


def _level_anchors_np(hf, wf, stride):
    base = _base_anchors_np()
    sx = (np.arange(wf) * stride).astype(np.float32)
    sy = (np.arange(hf) * stride).astype(np.float32)
    shy, shx = np.meshgrid(sy, sx, indexing='ij')
    shifts = np.stack([shx, shy, shx, shy], axis=-1).reshape(-1, 1, 4)
    return (shifts + base[None]).reshape(-1, 4).astype(np.float32)


def _rpn_head_kernel(xm, xc, xp, wk, bk, wc, bc, wr, br, o_out, d_out, scr,
                     *, TH, W):
    i = pl.program_id(1)
    nH = pl.num_programs(1)
    zrow = jnp.zeros((1, W, C), jnp.float32)
    top = jnp.where(i > 0, xm[0, TH - 1:TH], zrow)
    bot = jnp.where(i < nH - 1, xp[0, 0:1], zrow)
    zcol = jnp.zeros((TH + 2, 1, C), jnp.float32)
    scr[:, 0:1, :] = zcol
    scr[:, W + 1:W + 2, :] = zcol
    scr[0:1, 1:W + 1, :] = top
    scr[1:TH + 1, 1:W + 1, :] = xc[0]
    scr[TH + 1:TH + 2, 1:W + 1, :] = bot

    acc = jnp.zeros((TH * W, C), jnp.float32)
    for dy in range(3):
        for dx in range(3):
            xs = scr[dy:dy + TH, dx:dx + W, :].reshape(TH * W, C)
            acc = acc + jnp.dot(xs, wk[dy, dx],
                                preferred_element_type=jnp.float32)
    t = jnp.maximum(acc + bk[0], 0.0)
    o = jnp.dot(t, wc[...], preferred_element_type=jnp.float32) + bc[0]
    d = jnp.dot(t, wr[...], preferred_element_type=jnp.float32) + br[0]
    o_out[0] = o.reshape(TH, W, A)
    d_out[0] = d.reshape(TH, W, 4 * A)


_LEVEL_TH = {208: 8, 104: 8, 52: 13, 26: 26, 13: 13}


@functools.partial(jax.jit, static_argnames=("H", "W"))
def _rpn_head_level(x, wk, bk, wc, bc, wr, br, *, H, W):
    B = x.shape[0]
    TH = _LEVEL_TH[H]
    nH = H // TH
    kfn = functools.partial(_rpn_head_kernel, TH=TH, W=W)
    o, d = pl.pallas_call(
        kfn,
        grid=(B, nH),
        in_specs=[
            pl.BlockSpec((1, TH, W, C), lambda b, i: (b, jnp.maximum(i - 1, 0), 0, 0)),
            pl.BlockSpec((1, TH, W, C), lambda b, i: (b, i, 0, 0)),
            pl.BlockSpec((1, TH, W, C), lambda b, i: (b, jnp.minimum(i + 1, nH - 1), 0, 0)),
            pl.BlockSpec((3, 3, C, C), lambda b, i: (0, 0, 0, 0)),
            pl.BlockSpec((1, C), lambda b, i: (0, 0)),
            pl.BlockSpec((C, A), lambda b, i: (0, 0)),
            pl.BlockSpec((1, A), lambda b, i: (0, 0)),
            pl.BlockSpec((C, 4 * A), lambda b, i: (0, 0)),
            pl.BlockSpec((1, 4 * A), lambda b, i: (0, 0)),
        ],
        out_specs=[
            pl.BlockSpec((1, TH, W, A), lambda b, i: (b, i, 0, 0)),
            pl.BlockSpec((1, TH, W, 4 * A), lambda b, i: (b, i, 0, 0)),
        ],
        out_shape=[
            jax.ShapeDtypeStruct((B, H, W, A), jnp.float32),
            jax.ShapeDtypeStruct((B, H, W, 4 * A), jnp.float32),
        ],
        scratch_shapes=[pltpu.VMEM((TH + 2, W + 2, C), jnp.float32)],
        compiler_params=pltpu.CompilerParams(
            dimension_semantics=("parallel", "arbitrary")),
    )(x, x, x, wk, bk, wc, bc, wr, br)
    return o, d


def _decode(deltas, boxes, weights):
    wx, wy, ww, wh = weights
    widths = boxes[:, 2] - boxes[:, 0]
    heights = boxes[:, 3] - boxes[:, 1]
    cx = boxes[:, 0] + 0.5 * widths
    cy = boxes[:, 1] + 0.5 * heights
    dx = deltas[:, 0] / wx
    dy = deltas[:, 1] / wy
    dw = jnp.minimum(deltas[:, 2] / ww, LOG_CLAMP)
    dh = jnp.minimum(deltas[:, 3] / wh, LOG_CLAMP)
    pcx = dx * widths + cx
    pcy = dy * heights + cy
    pw = jnp.exp(dw) * widths
    ph = jnp.exp(dh) * heights
    return jnp.stack([pcx - 0.5 * pw, pcy - 0.5 * ph,
                      pcx + 0.5 * pw, pcy + 0.5 * ph], axis=1)


def _clip_boxes(boxes):
    x = jnp.clip(boxes[:, 0::2], 0.0, W_IMG)
    y = jnp.clip(boxes[:, 1::2], 0.0, H_IMG)
    return jnp.stack([x[:, 0], y[:, 0], x[:, 1], y[:, 1]], axis=1)


def _nms_keep(boxes, scores, thresh):
    boxes = lax.stop_gradient(boxes)
    scores = lax.stop_gradient(scores)
    n = boxes.shape[0]
    order = jnp.argsort(-scores)
    b = boxes[order]
    areas = (b[:, 2] - b[:, 0]) * (b[:, 3] - b[:, 1])
    idx = jnp.arange(n)

    def body(i, supp):
        xx1 = jnp.maximum(b[i, 0], b[:, 0])
        yy1 = jnp.maximum(b[i, 1], b[:, 1])
        xx2 = jnp.minimum(b[i, 2], b[:, 2])
        yy2 = jnp.minimum(b[i, 3], b[:, 3])
        inter = jnp.clip(xx2 - xx1, 0.0) * jnp.clip(yy2 - yy1, 0.0)
        iou = inter / (areas[i] + areas - inter + 1e-9)
        return supp | ((iou > thresh) & (idx > i) & (~supp[i]))

    supp = lax.fori_loop(0, n, body, jnp.zeros(n, dtype=bool))
    return jnp.zeros(n, dtype=bool).at[order].set(~supp)


def _rpn_proposals_single(objs, dels, ancs):
    boxes_l, scores_l, lvl_l = [], [], []
    for l in range(len(objs)):
        o, d, anc = objs[l], dels[l], ancs[l]
        k = min(PRE_NMS, o.shape[0])
        top_s, top_i = lax.top_k(o, k)
        bx = _clip_boxes(_decode(d[top_i], anc[top_i], (1.0, 1.0, 1.0, 1.0)))
        boxes_l.append(bx)
        scores_l.append(jax.nn.sigmoid(top_s))
        lvl_l.append(jnp.full((k,), float(l), jnp.float32))
    boxes = jnp.concatenate(boxes_l)
    scores = jnp.concatenate(scores_l)
    lvls = jnp.concatenate(lvl_l)
    valid = ((boxes[:, 2] - boxes[:, 0] >= 1e-3) &
             (boxes[:, 3] - boxes[:, 1] >= 1e-3))
    scores = jnp.where(valid, scores, -jnp.inf)
    off = lvls[:, None] * (max(H_IMG, W_IMG) + 2.0)
    keep = _nms_keep(boxes + off, scores, RPN_NMS_TH)
    final = jnp.where(keep, scores, -jnp.inf)
    top_s, top_i = lax.top_k(final, POST_NMS)
    return boxes[top_i], top_s


def _roi_align_one(feat, roi, scale):
    Cc, H, W = feat.shape
    x1, y1, x2, y2 = roi[0] * scale, roi[1] * scale, roi[2] * scale, roi[3] * scale
    rw = jnp.maximum(x2 - x1, 1.0)
    rh = jnp.maximum(y2 - y1, 1.0)
    g = POOL * SAMPLING
    xs = x1 + (jnp.arange(g) + 0.5) * (rw / POOL / SAMPLING)
    ys = y1 + (jnp.arange(g) + 0.5) * (rh / POOL / SAMPLING)
    yy, xx = jnp.meshgrid(ys, xs, indexing='ij')
    yy = yy.reshape(-1)
    xx = xx.reshape(-1)
    valid = (yy > -1.0) & (yy < H) & (xx > -1.0) & (xx < W)
    y = jnp.clip(yy, 0.0, H - 1.0)
    x = jnp.clip(xx, 0.0, W - 1.0)
    y0 = jnp.floor(y).astype(jnp.int32)
    x0 = jnp.floor(x).astype(jnp.int32)
    y1i = jnp.minimum(y0 + 1, H - 1)
    x1i = jnp.minimum(x0 + 1, W - 1)
    ly = y - y0
    lx = x - x0
    v = (feat[:, y0, x0] * (1 - ly) * (1 - lx) + feat[:, y0, x1i] * (1 - ly) * lx
         + feat[:, y1i, x0] * ly * (1 - lx) + feat[:, y1i, x1i] * ly * lx)
    v = v * valid
    return v.reshape(Cc, POOL, SAMPLING, POOL, SAMPLING).mean(axis=(2, 4))


def _pool_rois(feats, rois):
    area = jnp.clip(rois[:, 2] - rois[:, 0], 0.0) * jnp.clip(rois[:, 3] - rois[:, 1], 0.0)
    lvl = jnp.floor(4.0 + jnp.log2(jnp.sqrt(area) / CANONICAL + 1e-6))
    lvl = jnp.clip(lvl, 2.0, 5.0).astype(jnp.int32) - 2
    pooled = jnp.stack([
        jax.vmap(_roi_align_one, in_axes=(None, 0, None))(feats[l], rois, 1.0 / STRIDES[l])
        for l in range(4)])
    return pooled[lvl, jnp.arange(rois.shape[0])]


def _roi_heads_single(feats, proposals, prop_scores,
                      fc1_w, fc1_b, fc2_w, fc2_b, cls_w, cls_b, reg_w, reg_b):
    x = _pool_rois(feats, proposals).reshape(POST_NMS, -1)
    x = jax.nn.relu(x @ fc1_w + fc1_b)
    x = jax.nn.relu(x @ fc2_w + fc2_b)
    logits = x @ cls_w + cls_b
    deltas = x @ reg_w + reg_b
    scores = jax.nn.softmax(logits, axis=-1)[:, 1]
    boxes = _clip_boxes(_decode(deltas[:, 4:8], proposals, BBOX_W))
    valid = jnp.isfinite(prop_scores)
    valid &= (boxes[:, 2] - boxes[:, 0] >= 1e-2) & (boxes[:, 3] - boxes[:, 1] >= 1e-2)
    valid &= scores > SCORE_TH
    s = jnp.where(valid, scores, -jnp.inf)
    keep = _nms_keep(boxes, s, BOX_NMS_TH) & valid
    return jnp.where(keep[:, None], boxes, 0.0), jnp.where(keep, scores, 0.0)


def kernel(feats0, feats1, feats2, feats3, feats4,
           rpn_conv_w, rpn_conv_b, rpn_cls_w, rpn_cls_b, rpn_reg_w, rpn_reg_b,
           fc1_w, fc1_b, fc2_w, fc2_b, cls_w, cls_b, reg_w, reg_b):
    feats = (feats0, feats1, feats2, feats3, feats4)
    B = feats0.shape[0]

    wk = rpn_conv_w.transpose(2, 3, 1, 0)
    bk = rpn_conv_b.reshape(1, C)
    wc = rpn_cls_w.reshape(A, C).T
    bc = rpn_cls_b.reshape(1, A)
    wr = rpn_reg_w.reshape(4 * A, C).T
    br = rpn_reg_b.reshape(1, 4 * A)

    objs, dels, ancs = [], [], []
    for l, f in enumerate(feats):
        _, _, Hf, Wf = f.shape
        x = f.transpose(0, 2, 3, 1)
        o, d = _rpn_head_level(x, wk, bk, wc, bc, wr, br, H=Hf, W=Wf)
        objs.append(o.reshape(B, -1))
        dels.append(d.reshape(B, -1, 4))
        ancs.append(jnp.asarray(_level_anchors_np(Hf, Wf, STRIDES[l])))

    proposals, prop_scores = jax.vmap(_rpn_proposals_single, in_axes=(0, 0, None))(
        tuple(objs), tuple(dels), tuple(ancs))
    boxes, scores = jax.vmap(_roi_heads_single, in_axes=(0, 0, 0) + (None,) * 8)(
        (feats0, feats1, feats2, feats3), proposals, prop_scores,
        fc1_w, fc1_b, fc2_w, fc2_b, cls_w, cls_b, reg_w, reg_b)
    return boxes, scores


# speedup vs baseline: 1.1537x; 1.0592x over previous
import functools

import numpy as np
import jax
import jax.numpy as jnp
from jax import lax
from jax.experimental import pallas as pl
from jax.experimental.pallas import tpu as pltpu

H_IMG, W_IMG = 832.0, 640.0
STRIDES = (4, 8, 16, 32, 64)
SIZES = (64.0, 128.0, 256.0, 512.0, 1024.0)
RATIOS = (0.04, 0.05, 0.08, 0.15)
A = len(SIZES) * len(RATIOS)
PRE_NMS = 100
POST_NMS = 80
RPN_NMS_TH = 0.7
BOX_NMS_TH = 0.5
SCORE_TH = 0.05
POOL = 7
SAMPLING = 2
CANONICAL = 224.0
BBOX_W = (10.0, 10.0, 5.0, 5.0)
LOG_CLAMP = float(np.log(1000.0 / 16.0))

C = 256


def _base_anchors_np():
    scales = np.array(SIZES, np.float32)
    ratios = np.array(RATIOS, np.float32)
    h_r = np.sqrt(ratios)
    w_r = 1.0 / h_r
    ws = (w_r[:, None] * scales[None, :]).reshape(-1)
    hs = (h_r[:, None] * scales[None, :]).reshape(-1)
    return np.round(np.stack([-ws, -hs, ws, hs], axis=1) / 2.0)


def _level_anchors_np(hf, wf, stride):
    base = _base_anchors_np()
    sx = (np.arange(wf) * stride).astype(np.float32)
    sy = (np.arange(hf) * stride).astype(np.float32)
    shy, shx = np.meshgrid(sy, sx, indexing='ij')
    shifts = np.stack([shx, shy, shx, shy], axis=-1).reshape(-1, 1, 4)
    return (shifts + base[None]).reshape(-1, 4).astype(np.float32)


def _rpn_head_kernel(xm, xc, xp, wk, bk, wc, bc, wr, br, o_out, d_out, scr,
                     *, TH, W):
    i = pl.program_id(1)
    nH = pl.num_programs(1)
    zrow = jnp.zeros((1, W, C), jnp.float32)
    top = jnp.where(i > 0, xm[0, TH - 1:TH], zrow)
    bot = jnp.where(i < nH - 1, xp[0, 0:1], zrow)
    zcol = jnp.zeros((TH + 2, 1, C), jnp.float32)
    scr[:, 0:1, :] = zcol
    scr[:, W + 1:W + 2, :] = zcol
    scr[0:1, 1:W + 1, :] = top
    scr[1:TH + 1, 1:W + 1, :] = xc[0]
    scr[TH + 1:TH + 2, 1:W + 1, :] = bot

    acc = jnp.zeros((TH * W, C), jnp.float32)
    for dy in range(3):
        for dx in range(3):
            xs = scr[dy:dy + TH, dx:dx + W, :].reshape(TH * W, C)
            acc = acc + jnp.dot(xs, wk[dy, dx],
                                preferred_element_type=jnp.float32)
    t = jnp.maximum(acc + bk[0], 0.0)
    o = jnp.dot(t, wc[...], preferred_element_type=jnp.float32) + bc[0]
    d = jnp.dot(t, wr[...], preferred_element_type=jnp.float32) + br[0]
    o_out[0] = o.reshape(TH, W, A)
    d_out[0] = d.reshape(TH, W, 4 * A)


_LEVEL_TH = {208: 8, 104: 8, 52: 13, 26: 26, 13: 13}


@functools.partial(jax.jit, static_argnames=("H", "W"))
def _rpn_head_level(x, wk, bk, wc, bc, wr, br, *, H, W):
    B = x.shape[0]
    TH = _LEVEL_TH[H]
    nH = H // TH
    kfn = functools.partial(_rpn_head_kernel, TH=TH, W=W)
    o, d = pl.pallas_call(
        kfn,
        grid=(B, nH),
        in_specs=[
            pl.BlockSpec((1, TH, W, C), lambda b, i: (b, jnp.maximum(i - 1, 0), 0, 0)),
            pl.BlockSpec((1, TH, W, C), lambda b, i: (b, i, 0, 0)),
            pl.BlockSpec((1, TH, W, C), lambda b, i: (b, jnp.minimum(i + 1, nH - 1), 0, 0)),
            pl.BlockSpec((3, 3, C, C), lambda b, i: (0, 0, 0, 0)),
            pl.BlockSpec((1, C), lambda b, i: (0, 0)),
            pl.BlockSpec((C, A), lambda b, i: (0, 0)),
            pl.BlockSpec((1, A), lambda b, i: (0, 0)),
            pl.BlockSpec((C, 4 * A), lambda b, i: (0, 0)),
            pl.BlockSpec((1, 4 * A), lambda b, i: (0, 0)),
        ],
        out_specs=[
            pl.BlockSpec((1, TH, W, A), lambda b, i: (b, i, 0, 0)),
            pl.BlockSpec((1, TH, W, 4 * A), lambda b, i: (b, i, 0, 0)),
        ],
        out_shape=[
            jax.ShapeDtypeStruct((B, H, W, A), jnp.float32),
            jax.ShapeDtypeStruct((B, H, W, 4 * A), jnp.float32),
        ],
        scratch_shapes=[pltpu.VMEM((TH + 2, W + 2, C), jnp.float32)],
        compiler_params=pltpu.CompilerParams(
            dimension_semantics=("parallel", "arbitrary")),
    )(x, x, x, wk, bk, wc, bc, wr, br)
    return o, d


def _decode(deltas, boxes, weights):
    wx, wy, ww, wh = weights
    widths = boxes[:, 2] - boxes[:, 0]
    heights = boxes[:, 3] - boxes[:, 1]
    cx = boxes[:, 0] + 0.5 * widths
    cy = boxes[:, 1] + 0.5 * heights
    dx = deltas[:, 0] / wx
    dy = deltas[:, 1] / wy
    dw = jnp.minimum(deltas[:, 2] / ww, LOG_CLAMP)
    dh = jnp.minimum(deltas[:, 3] / wh, LOG_CLAMP)
    pcx = dx * widths + cx
    pcy = dy * heights + cy
    pw = jnp.exp(dw) * widths
    ph = jnp.exp(dh) * heights
    return jnp.stack([pcx - 0.5 * pw, pcy - 0.5 * ph,
                      pcx + 0.5 * pw, pcy + 0.5 * ph], axis=1)


def _clip_boxes(boxes):
    x = jnp.clip(boxes[:, 0::2], 0.0, W_IMG)
    y = jnp.clip(boxes[:, 1::2], 0.0, H_IMG)
    return jnp.stack([x[:, 0], y[:, 0], x[:, 1], y[:, 1]], axis=1)


def _nms_fix_kernel(b_ref, bt_ref, keep_ref, mref, *, N, thresh):
    x1c = jnp.broadcast_to(b_ref[0, :, 0:1], (N, N))
    y1c = jnp.broadcast_to(b_ref[0, :, 1:2], (N, N))
    x2c = jnp.broadcast_to(b_ref[0, :, 2:3], (N, N))
    y2c = jnp.broadcast_to(b_ref[0, :, 3:4], (N, N))
    x1r = jnp.broadcast_to(bt_ref[0, 0:1, :], (N, N))
    y1r = jnp.broadcast_to(bt_ref[0, 1:2, :], (N, N))
    x2r = jnp.broadcast_to(bt_ref[0, 2:3, :], (N, N))
    y2r = jnp.broadcast_to(bt_ref[0, 3:4, :], (N, N))
    areac = (x2c - x1c) * (y2c - y1c)
    arear = (x2r - x1r) * (y2r - y1r)
    inter = (jnp.clip(jnp.minimum(x2c, x2r) - jnp.maximum(x1c, x1r), 0.0)
             * jnp.clip(jnp.minimum(y2c, y2r) - jnp.maximum(y1c, y1r), 0.0))
    iou = inter / (areac + arear - inter + 1e-9)
    ii = lax.broadcasted_iota(jnp.int32, (N, N), 0)
    jj = lax.broadcasted_iota(jnp.int32, (N, N), 1)
    mref[...] = jnp.where((iou > thresh) & (jj > ii), 1.0, 0.0)

    def cond(c):
        _, ch, t = c
        return ch & (t < N + 2)

    def body(c):
        k, _, t = c
        sup = jnp.dot(k, mref[...], preferred_element_type=jnp.float32)
        nk = jnp.where(sup > 0.0, 0.0, 1.0)
        ch = jnp.sum(jnp.abs(nk - k)) > 0.0
        return nk, ch, t + 1

    k0 = jnp.ones((1, N), jnp.float32)
    kf, _, _ = lax.while_loop(cond, body, (k0, jnp.bool_(True), jnp.int32(0)))
    keep_ref[0] = kf


def _nms_pallas(boxes_sorted, thresh):
    B, N, _ = boxes_sorted.shape
    kfn = functools.partial(_nms_fix_kernel, N=N, thresh=thresh)
    keep = pl.pallas_call(
        kfn,
        grid=(B,),
        in_specs=[
            pl.BlockSpec((1, N, 4), lambda b: (b, 0, 0)),
            pl.BlockSpec((1, 4, N), lambda b: (b, 0, 0)),
        ],
        out_specs=pl.BlockSpec((1, 1, N), lambda b: (b, 0, 0)),
        out_shape=jax.ShapeDtypeStruct((B, 1, N), jnp.float32),
        scratch_shapes=[pltpu.VMEM((N, N), jnp.float32)],
        compiler_params=pltpu.CompilerParams(
            dimension_semantics=("parallel",)),
    )(boxes_sorted, boxes_sorted.transpose(0, 2, 1))
    return keep[:, 0, :]


def _roi_align_one(feat, roi, scale):
    Cc, H, W = feat.shape
    x1, y1, x2, y2 = roi[0] * scale, roi[1] * scale, roi[2] * scale, roi[3] * scale
    rw = jnp.maximum(x2 - x1, 1.0)
    rh = jnp.maximum(y2 - y1, 1.0)
    g = POOL * SAMPLING
    xs = x1 + (jnp.arange(g) + 0.5) * (rw / POOL / SAMPLING)
    ys = y1 + (jnp.arange(g) + 0.5) * (rh / POOL / SAMPLING)
    yy, xx = jnp.meshgrid(ys, xs, indexing='ij')
    yy = yy.reshape(-1)
    xx = xx.reshape(-1)
    valid = (yy > -1.0) & (yy < H) & (xx > -1.0) & (xx < W)
    y = jnp.clip(yy, 0.0, H - 1.0)
    x = jnp.clip(xx, 0.0, W - 1.0)
    y0 = jnp.floor(y).astype(jnp.int32)
    x0 = jnp.floor(x).astype(jnp.int32)
    y1i = jnp.minimum(y0 + 1, H - 1)
    x1i = jnp.minimum(x0 + 1, W - 1)
    ly = y - y0
    lx = x - x0
    v = (feat[:, y0, x0] * (1 - ly) * (1 - lx) + feat[:, y0, x1i] * (1 - ly) * lx
         + feat[:, y1i, x0] * ly * (1 - lx) + feat[:, y1i, x1i] * ly * lx)
    v = v * valid
    return v.reshape(Cc, POOL, SAMPLING, POOL, SAMPLING).mean(axis=(2, 4))


def _pool_rois(feats, rois):
    area = jnp.clip(rois[:, 2] - rois[:, 0], 0.0) * jnp.clip(rois[:, 3] - rois[:, 1], 0.0)
    lvl = jnp.floor(4.0 + jnp.log2(jnp.sqrt(area) / CANONICAL + 1e-6))
    lvl = jnp.clip(lvl, 2.0, 5.0).astype(jnp.int32) - 2
    pooled = jnp.stack([
        jax.vmap(_roi_align_one, in_axes=(None, 0, None))(feats[l], rois, 1.0 / STRIDES[l])
        for l in range(4)])
    return pooled[lvl, jnp.arange(rois.shape[0])]


def _take2(x, idx):
    if x.ndim == 3:
        return jnp.take_along_axis(x, idx[:, :, None], axis=1)
    return jnp.take_along_axis(x, idx, axis=1)


def _rpn_proposals(objs, dels, ancs):
    B = objs[0].shape[0]
    boxes_l, scores_l, lvl_l = [], [], []
    for l in range(len(objs)):
        o, d, anc = objs[l], dels[l], ancs[l]
        k = min(PRE_NMS, o.shape[1])
        top_s, top_i = lax.top_k(o, k)
        dk = _take2(d, top_i).reshape(B * k, 4)
        ak = anc[top_i].reshape(B * k, 4)
        bx = _clip_boxes(_decode(dk, ak, (1.0, 1.0, 1.0, 1.0))).reshape(B, k, 4)
        boxes_l.append(bx)
        scores_l.append(jax.nn.sigmoid(top_s))
        lvl_l.append(jnp.full((B, k), float(l), jnp.float32))
    boxes = jnp.concatenate(boxes_l, axis=1)
    scores = jnp.concatenate(scores_l, axis=1)
    lvls = jnp.concatenate(lvl_l, axis=1)
    valid = ((boxes[:, :, 2] - boxes[:, :, 0] >= 1e-3) &
             (boxes[:, :, 3] - boxes[:, :, 1] >= 1e-3))
    scores = jnp.where(valid, scores, -jnp.inf)
    off = lvls[:, :, None] * (max(H_IMG, W_IMG) + 2.0)
    N = boxes.shape[1]
    NP = 512
    pb = jnp.zeros((B, NP - N, 4), jnp.float32)
    boxes_p = jnp.concatenate([boxes, pb], axis=1)
    off_p = jnp.concatenate([boxes + off, pb], axis=1)
    scores_p = jnp.concatenate(
        [scores, jnp.full((B, NP - N), -jnp.inf, jnp.float32)], axis=1)
    order = jnp.argsort(-scores_p, axis=1)
    keep_s = _nms_pallas(_take2(off_p, order), RPN_NMS_TH)
    final_s = jnp.where(keep_s > 0, _take2(scores_p, order), -jnp.inf)
    top_s, top_i = lax.top_k(final_s, POST_NMS)
    proposals = _take2(_take2(boxes_p, order), top_i)
    return proposals, top_s


def _roi_scores_single(feats, proposals,
                       fc1_w, fc1_b, fc2_w, fc2_b, cls_w, cls_b, reg_w, reg_b):
    x = _pool_rois(feats, proposals).reshape(POST_NMS, -1)
    x = jax.nn.relu(x @ fc1_w + fc1_b)
    x = jax.nn.relu(x @ fc2_w + fc2_b)
    logits = x @ cls_w + cls_b
    deltas = x @ reg_w + reg_b
    scores = jax.nn.softmax(logits, axis=-1)[:, 1]
    boxes = _clip_boxes(_decode(deltas[:, 4:8], proposals, BBOX_W))
    return boxes, scores


def _roi_heads(feats, proposals, prop_scores,
               fc1_w, fc1_b, fc2_w, fc2_b, cls_w, cls_b, reg_w, reg_b):
    B = proposals.shape[0]
    boxes, scores = jax.vmap(
        _roi_scores_single, in_axes=(0, 0) + (None,) * 8)(
        feats, proposals, fc1_w, fc1_b, fc2_w, fc2_b, cls_w, cls_b, reg_w, reg_b)
    valid = jnp.isfinite(prop_scores)
    valid &= (boxes[:, :, 2] - boxes[:, :, 0] >= 1e-2) & (boxes[:, :, 3] - boxes[:, :, 1] >= 1e-2)
    valid &= scores > SCORE_TH
    s = jnp.where(valid, scores, -jnp.inf)
    N = boxes.shape[1]
    NP = 128
    boxes_p = jnp.concatenate(
        [boxes, jnp.zeros((B, NP - N, 4), jnp.float32)], axis=1)
    s_p = jnp.concatenate(
        [s, jnp.full((B, NP - N), -jnp.inf, jnp.float32)], axis=1)
    order = jnp.argsort(-s_p, axis=1)
    keep_sorted = _nms_pallas(_take2(boxes_p, order), BOX_NMS_TH)
    keep = jnp.zeros((B, NP), bool).at[
        jnp.arange(B)[:, None], order].set(keep_sorted > 0)[:, :N]
    keep &= valid
    return jnp.where(keep[:, :, None], boxes, 0.0), jnp.where(keep, scores, 0.0)


def kernel(feats0, feats1, feats2, feats3, feats4,
           rpn_conv_w, rpn_conv_b, rpn_cls_w, rpn_cls_b, rpn_reg_w, rpn_reg_b,
           fc1_w, fc1_b, fc2_w, fc2_b, cls_w, cls_b, reg_w, reg_b):
    feats = (feats0, feats1, feats2, feats3, feats4)
    B = feats0.shape[0]

    wk = rpn_conv_w.transpose(2, 3, 1, 0)
    bk = rpn_conv_b.reshape(1, C)
    wc = rpn_cls_w.reshape(A, C).T
    bc = rpn_cls_b.reshape(1, A)
    wr = rpn_reg_w.reshape(4 * A, C).T
    br = rpn_reg_b.reshape(1, 4 * A)

    objs, dels, ancs = [], [], []
    for l, f in enumerate(feats):
        _, _, Hf, Wf = f.shape
        x = f.transpose(0, 2, 3, 1)
        o, d = _rpn_head_level(x, wk, bk, wc, bc, wr, br, H=Hf, W=Wf)
        objs.append(o.reshape(B, -1))
        dels.append(d.reshape(B, -1, 4))
        ancs.append(jnp.asarray(_level_anchors_np(Hf, Wf, STRIDES[l])))

    proposals, prop_scores = _rpn_proposals(objs, dels, ancs)
    boxes, scores = _roi_heads(
        (feats0, feats1, feats2, feats3), proposals, prop_scores,
        fc1_w, fc1_b, fc2_w, fc2_b, cls_w, cls_b, reg_w, reg_b)
    return boxes, scores


# speedup vs baseline: 3.8522x; 3.3391x over previous
import functools

import numpy as np
import jax
import jax.numpy as jnp
from jax import lax
from jax.experimental import pallas as pl
from jax.experimental.pallas import tpu as pltpu

H_IMG, W_IMG = 832.0, 640.0
STRIDES = (4, 8, 16, 32, 64)
SIZES = (64.0, 128.0, 256.0, 512.0, 1024.0)
RATIOS = (0.04, 0.05, 0.08, 0.15)
A = len(SIZES) * len(RATIOS)
PRE_NMS = 100
POST_NMS = 80
RPN_NMS_TH = 0.7
BOX_NMS_TH = 0.5
SCORE_TH = 0.05
POOL = 7
SAMPLING = 2
CANONICAL = 224.0
BBOX_W = (10.0, 10.0, 5.0, 5.0)
LOG_CLAMP = float(np.log(1000.0 / 16.0))

C = 256


def _base_anchors_np():
    scales = np.array(SIZES, np.float32)
    ratios = np.array(RATIOS, np.float32)
    h_r = np.sqrt(ratios)
    w_r = 1.0 / h_r
    ws = (w_r[:, None] * scales[None, :]).reshape(-1)
    hs = (h_r[:, None] * scales[None, :]).reshape(-1)
    return np.round(np.stack([-ws, -hs, ws, hs], axis=1) / 2.0)


def _level_anchors_np(hf, wf, stride):
    base = _base_anchors_np()
    sx = (np.arange(wf) * stride).astype(np.float32)
    sy = (np.arange(hf) * stride).astype(np.float32)
    shy, shx = np.meshgrid(sy, sx, indexing='ij')
    shifts = np.stack([shx, shy, shx, shy], axis=-1).reshape(-1, 1, 4)
    return (shifts + base[None]).reshape(-1, 4).astype(np.float32)


def _rpn_head_kernel(xm, xc, xp, wk, bk, wc, bc, wr, br, o_out, d_out, scr,
                     *, TH, W):
    i = pl.program_id(1)
    nH = pl.num_programs(1)
    zrow = jnp.zeros((1, W, C), jnp.float32)
    top = jnp.where(i > 0, xm[0, TH - 1:TH], zrow)
    bot = jnp.where(i < nH - 1, xp[0, 0:1], zrow)
    zcol = jnp.zeros((TH + 2, 1, C), jnp.float32)
    scr[:, 0:1, :] = zcol
    scr[:, W + 1:W + 2, :] = zcol
    scr[0:1, 1:W + 1, :] = top
    scr[1:TH + 1, 1:W + 1, :] = xc[0]
    scr[TH + 1:TH + 2, 1:W + 1, :] = bot

    acc = jnp.zeros((TH * W, C), jnp.float32)
    for dy in range(3):
        for dx in range(3):
            xs = scr[dy:dy + TH, dx:dx + W, :].reshape(TH * W, C)
            acc = acc + jnp.dot(xs, wk[dy, dx],
                                preferred_element_type=jnp.float32)
    t = jnp.maximum(acc + bk[0], 0.0)
    o = jnp.dot(t, wc[...], preferred_element_type=jnp.float32) + bc[0]
    d = jnp.dot(t, wr[...], preferred_element_type=jnp.float32) + br[0]
    o_out[0] = o.reshape(TH, W, A)
    d_out[0] = d.reshape(TH, W, 4 * A)


_LEVEL_TH = {208: 8, 104: 8, 52: 13, 26: 26, 13: 13}


@functools.partial(jax.jit, static_argnames=("H", "W"))
def _rpn_head_level(x, wk, bk, wc, bc, wr, br, *, H, W):
    B = x.shape[0]
    TH = _LEVEL_TH[H]
    nH = H // TH
    kfn = functools.partial(_rpn_head_kernel, TH=TH, W=W)
    o, d = pl.pallas_call(
        kfn,
        grid=(B, nH),
        in_specs=[
            pl.BlockSpec((1, TH, W, C), lambda b, i: (b, jnp.maximum(i - 1, 0), 0, 0)),
            pl.BlockSpec((1, TH, W, C), lambda b, i: (b, i, 0, 0)),
            pl.BlockSpec((1, TH, W, C), lambda b, i: (b, jnp.minimum(i + 1, nH - 1), 0, 0)),
            pl.BlockSpec((3, 3, C, C), lambda b, i: (0, 0, 0, 0)),
            pl.BlockSpec((1, C), lambda b, i: (0, 0)),
            pl.BlockSpec((C, A), lambda b, i: (0, 0)),
            pl.BlockSpec((1, A), lambda b, i: (0, 0)),
            pl.BlockSpec((C, 4 * A), lambda b, i: (0, 0)),
            pl.BlockSpec((1, 4 * A), lambda b, i: (0, 0)),
        ],
        out_specs=[
            pl.BlockSpec((1, TH, W, A), lambda b, i: (b, i, 0, 0)),
            pl.BlockSpec((1, TH, W, 4 * A), lambda b, i: (b, i, 0, 0)),
        ],
        out_shape=[
            jax.ShapeDtypeStruct((B, H, W, A), jnp.float32),
            jax.ShapeDtypeStruct((B, H, W, 4 * A), jnp.float32),
        ],
        scratch_shapes=[pltpu.VMEM((TH + 2, W + 2, C), jnp.float32)],
        compiler_params=pltpu.CompilerParams(
            dimension_semantics=("parallel", "arbitrary")),
    )(x, x, x, wk, bk, wc, bc, wr, br)
    return o, d


def _decode(deltas, boxes, weights):
    wx, wy, ww, wh = weights
    widths = boxes[:, 2] - boxes[:, 0]
    heights = boxes[:, 3] - boxes[:, 1]
    cx = boxes[:, 0] + 0.5 * widths
    cy = boxes[:, 1] + 0.5 * heights
    dx = deltas[:, 0] / wx
    dy = deltas[:, 1] / wy
    dw = jnp.minimum(deltas[:, 2] / ww, LOG_CLAMP)
    dh = jnp.minimum(deltas[:, 3] / wh, LOG_CLAMP)
    pcx = dx * widths + cx
    pcy = dy * heights + cy
    pw = jnp.exp(dw) * widths
    ph = jnp.exp(dh) * heights
    return jnp.stack([pcx - 0.5 * pw, pcy - 0.5 * ph,
                      pcx + 0.5 * pw, pcy + 0.5 * ph], axis=1)


def _clip_boxes(boxes):
    x = jnp.clip(boxes[:, 0::2], 0.0, W_IMG)
    y = jnp.clip(boxes[:, 1::2], 0.0, H_IMG)
    return jnp.stack([x[:, 0], y[:, 0], x[:, 1], y[:, 1]], axis=1)


def _nms_fix_kernel(b_ref, bt_ref, keep_ref, mref, *, N, thresh):
    x1c = jnp.broadcast_to(b_ref[0, :, 0:1], (N, N))
    y1c = jnp.broadcast_to(b_ref[0, :, 1:2], (N, N))
    x2c = jnp.broadcast_to(b_ref[0, :, 2:3], (N, N))
    y2c = jnp.broadcast_to(b_ref[0, :, 3:4], (N, N))
    x1r = jnp.broadcast_to(bt_ref[0, 0:1, :], (N, N))
    y1r = jnp.broadcast_to(bt_ref[0, 1:2, :], (N, N))
    x2r = jnp.broadcast_to(bt_ref[0, 2:3, :], (N, N))
    y2r = jnp.broadcast_to(bt_ref[0, 3:4, :], (N, N))
    areac = (x2c - x1c) * (y2c - y1c)
    arear = (x2r - x1r) * (y2r - y1r)
    inter = (jnp.clip(jnp.minimum(x2c, x2r) - jnp.maximum(x1c, x1r), 0.0)
             * jnp.clip(jnp.minimum(y2c, y2r) - jnp.maximum(y1c, y1r), 0.0))
    iou = inter / (areac + arear - inter + 1e-9)
    ii = lax.broadcasted_iota(jnp.int32, (N, N), 0)
    jj = lax.broadcasted_iota(jnp.int32, (N, N), 1)
    mref[...] = jnp.where((iou > thresh) & (jj > ii), 1.0, 0.0)

    def cond(c):
        _, ch, t = c
        return ch & (t < N + 2)

    def body(c):
        k, _, t = c
        sup = jnp.dot(k, mref[...], preferred_element_type=jnp.float32)
        nk = jnp.where(sup > 0.0, 0.0, 1.0)
        ch = jnp.sum(jnp.abs(nk - k)) > 0.0
        return nk, ch, t + 1

    k0 = jnp.ones((1, N), jnp.float32)
    kf, _, _ = lax.while_loop(cond, body, (k0, jnp.bool_(True), jnp.int32(0)))
    keep_ref[0] = kf


def _nms_pallas(boxes_sorted, thresh):
    B, N, _ = boxes_sorted.shape
    kfn = functools.partial(_nms_fix_kernel, N=N, thresh=thresh)
    keep = pl.pallas_call(
        kfn,
        grid=(B,),
        in_specs=[
            pl.BlockSpec((1, N, 4), lambda b: (b, 0, 0)),
            pl.BlockSpec((1, 4, N), lambda b: (b, 0, 0)),
        ],
        out_specs=pl.BlockSpec((1, 1, N), lambda b: (b, 0, 0)),
        out_shape=jax.ShapeDtypeStruct((B, 1, N), jnp.float32),
        scratch_shapes=[pltpu.VMEM((N, N), jnp.float32)],
        compiler_params=pltpu.CompilerParams(
            dimension_semantics=("parallel",)),
    )(boxes_sorted, boxes_sorted.transpose(0, 2, 1))
    return keep[:, 0, :]


def _pool_one(scr, out_ref, r, x1, y1, x2, y2, H, W):
    rw = jnp.maximum(x2 - x1, 1.0)
    rh = jnp.maximum(y2 - y1, 1.0)
    stepx = rw / float(POOL) / float(SAMPLING)
    stepy = rh / float(POOL) / float(SAMPLING)
    pr = lax.broadcasted_iota(jnp.int32, (POOL, W), 0).astype(jnp.float32)
    xi = lax.broadcasted_iota(jnp.int32, (POOL, W), 1).astype(jnp.float32)
    g = jnp.zeros((POOL, W), jnp.float32)
    for s in (0, 1):
        xs = x1 + (2.0 * pr + (s + 0.5)) * stepx
        xv = jnp.where((xs > -1.0) & (xs < float(W)), 1.0, 0.0)
        xc = jnp.clip(xs, 0.0, float(W - 1))
        x0 = jnp.floor(xc)
        x1i = jnp.minimum(x0 + 1.0, float(W - 1))
        lx = xc - x0
        g = (g + jnp.where(xi == x0, (1.0 - lx) * xv, 0.0)
             + jnp.where(xi == x1i, lx * xv, 0.0))
    rows = []
    for pp in range(POOL):
        racc = jnp.zeros((W, C), jnp.float32)
        for s in (0, 1):
            yy = y1 + (2.0 * pp + (s + 0.5)) * stepy
            yv = jnp.where((yy > -1.0) & (yy < float(H)), 1.0, 0.0)
            yc = jnp.clip(yy, 0.0, float(H - 1))
            y0f = jnp.floor(yc)
            ly = yc - y0f
            y0 = y0f.astype(jnp.int32)
            y1i = jnp.minimum(y0 + 1, H - 1)
            racc = racc + (yv * (1.0 - ly)) * scr[y0] + (yv * ly) * scr[y1i]
        rows.append(jnp.dot(g, racc, preferred_element_type=jnp.float32) * 0.25)
    out_ref[0, r] = jnp.concatenate(rows, axis=0)


def _roi_pool_kernel(f0, f1, f2, f3, rl, out, s0, s1, s2, s3, sems):
    b = pl.program_id(0)
    frefs = (f0, f1, f2, f3)
    scrs = (s0, s1, s2, s3)
    for l in range(4):
        pltpu.make_async_copy(frefs[l].at[b], scrs[l], sems.at[l]).start()
    for l in range(4):
        pltpu.make_async_copy(frefs[l].at[b], scrs[l], sems.at[l]).wait()

    def body(r, carry):
        lv = rl[0, r, 4]
        for l in range(4):
            @pl.when(lv == float(l))
            def _(l=l):
                sc = 1.0 / float(STRIDES[l])
                Hl, Wl = scrs[l].shape[0], scrs[l].shape[1]
                _pool_one(scrs[l], out, r,
                          rl[0, r, 0] * sc, rl[0, r, 1] * sc,
                          rl[0, r, 2] * sc, rl[0, r, 3] * sc, Hl, Wl)
        return carry

    lax.fori_loop(0, POST_NMS, body, jnp.int32(0))


def _roi_pool_pallas(xs_nhwc, rois):
    B = rois.shape[0]
    area = (jnp.clip(rois[:, :, 2] - rois[:, :, 0], 0.0)
            * jnp.clip(rois[:, :, 3] - rois[:, :, 1], 0.0))
    lvl = jnp.floor(4.0 + jnp.log2(jnp.sqrt(area) / CANONICAL + 1e-6))
    lvl = jnp.clip(lvl, 2.0, 5.0) - 2.0
    rl = jnp.concatenate(
        [rois, lvl[:, :, None], jnp.zeros((B, POST_NMS, 3), jnp.float32)],
        axis=-1)
    pooled = pl.pallas_call(
        _roi_pool_kernel,
        grid=(B,),
        in_specs=[
            pl.BlockSpec(memory_space=pl.ANY),
            pl.BlockSpec(memory_space=pl.ANY),
            pl.BlockSpec(memory_space=pl.ANY),
            pl.BlockSpec(memory_space=pl.ANY),
            pl.BlockSpec((1, POST_NMS, 8), lambda b: (b, 0, 0),
                         memory_space=pltpu.SMEM),
        ],
        out_specs=pl.BlockSpec((1, POST_NMS, POOL * POOL, C),
                               lambda b: (b, 0, 0, 0)),
        out_shape=jax.ShapeDtypeStruct((B, POST_NMS, POOL * POOL, C),
                                       jnp.float32),
        scratch_shapes=[
            pltpu.VMEM(xs_nhwc[0].shape[1:], jnp.float32),
            pltpu.VMEM(xs_nhwc[1].shape[1:], jnp.float32),
            pltpu.VMEM(xs_nhwc[2].shape[1:], jnp.float32),
            pltpu.VMEM(xs_nhwc[3].shape[1:], jnp.float32),
            pltpu.SemaphoreType.DMA((4,)),
        ],
        compiler_params=pltpu.CompilerParams(
            dimension_semantics=("parallel",)),
    )(*xs_nhwc, rl)
    return pooled


def _take2(x, idx):
    if x.ndim == 3:
        return jnp.take_along_axis(x, idx[:, :, None], axis=1)
    return jnp.take_along_axis(x, idx, axis=1)


def _rpn_proposals(objs, dels, ancs):
    B = objs[0].shape[0]
    boxes_l, scores_l, lvl_l = [], [], []
    for l in range(len(objs)):
        o, d, anc = objs[l], dels[l], ancs[l]
        k = min(PRE_NMS, o.shape[1])
        top_s, top_i = lax.top_k(o, k)
        dk = _take2(d, top_i).reshape(B * k, 4)
        ak = anc[top_i].reshape(B * k, 4)
        bx = _clip_boxes(_decode(dk, ak, (1.0, 1.0, 1.0, 1.0))).reshape(B, k, 4)
        boxes_l.append(bx)
        scores_l.append(jax.nn.sigmoid(top_s))
        lvl_l.append(jnp.full((B, k), float(l), jnp.float32))
    boxes = jnp.concatenate(boxes_l, axis=1)
    scores = jnp.concatenate(scores_l, axis=1)
    lvls = jnp.concatenate(lvl_l, axis=1)
    valid = ((boxes[:, :, 2] - boxes[:, :, 0] >= 1e-3) &
             (boxes[:, :, 3] - boxes[:, :, 1] >= 1e-3))
    scores = jnp.where(valid, scores, -jnp.inf)
    off = lvls[:, :, None] * (max(H_IMG, W_IMG) + 2.0)
    N = boxes.shape[1]
    NP = 512
    pb = jnp.zeros((B, NP - N, 4), jnp.float32)
    boxes_p = jnp.concatenate([boxes, pb], axis=1)
    off_p = jnp.concatenate([boxes + off, pb], axis=1)
    scores_p = jnp.concatenate(
        [scores, jnp.full((B, NP - N), -jnp.inf, jnp.float32)], axis=1)
    order = jnp.argsort(-scores_p, axis=1)
    keep_s = _nms_pallas(_take2(off_p, order), RPN_NMS_TH)
    final_s = jnp.where(keep_s > 0, _take2(scores_p, order), -jnp.inf)
    top_s, top_i = lax.top_k(final_s, POST_NMS)
    proposals = _take2(_take2(boxes_p, order), top_i)
    return proposals, top_s


def _roi_heads(xs_nhwc, proposals, prop_scores,
               fc1_w, fc1_b, fc2_w, fc2_b, cls_w, cls_b, reg_w, reg_b):
    B = proposals.shape[0]
    pooled = _roi_pool_pallas(xs_nhwc, proposals)
    x = jnp.einsum('bnpc,cpj->bnj', pooled,
                   fc1_w.reshape(C, POOL * POOL, fc1_w.shape[1]),
                   preferred_element_type=jnp.float32)
    x = jax.nn.relu(x + fc1_b)
    x = jax.nn.relu(x @ fc2_w + fc2_b)
    logits = x @ cls_w + cls_b
    deltas = x @ reg_w + reg_b
    scores = jax.nn.softmax(logits, axis=-1)[:, :, 1]
    boxes = _clip_boxes(
        _decode(deltas[:, :, 4:8].reshape(B * POST_NMS, 4),
                proposals.reshape(B * POST_NMS, 4), BBOX_W)
    ).reshape(B, POST_NMS, 4)
    valid = jnp.isfinite(prop_scores)
    valid &= (boxes[:, :, 2] - boxes[:, :, 0] >= 1e-2) & (boxes[:, :, 3] - boxes[:, :, 1] >= 1e-2)
    valid &= scores > SCORE_TH
    s = jnp.where(valid, scores, -jnp.inf)
    N = boxes.shape[1]
    NP = 128
    boxes_p = jnp.concatenate(
        [boxes, jnp.zeros((B, NP - N, 4), jnp.float32)], axis=1)
    s_p = jnp.concatenate(
        [s, jnp.full((B, NP - N), -jnp.inf, jnp.float32)], axis=1)
    order = jnp.argsort(-s_p, axis=1)
    keep_sorted = _nms_pallas(_take2(boxes_p, order), BOX_NMS_TH)
    keep = jnp.zeros((B, NP), bool).at[
        jnp.arange(B)[:, None], order].set(keep_sorted > 0)[:, :N]
    keep &= valid
    return jnp.where(keep[:, :, None], boxes, 0.0), jnp.where(keep, scores, 0.0)


def kernel(feats0, feats1, feats2, feats3, feats4,
           rpn_conv_w, rpn_conv_b, rpn_cls_w, rpn_cls_b, rpn_reg_w, rpn_reg_b,
           fc1_w, fc1_b, fc2_w, fc2_b, cls_w, cls_b, reg_w, reg_b):
    feats = (feats0, feats1, feats2, feats3, feats4)
    B = feats0.shape[0]

    wk = rpn_conv_w.transpose(2, 3, 1, 0)
    bk = rpn_conv_b.reshape(1, C)
    wc = rpn_cls_w.reshape(A, C).T
    bc = rpn_cls_b.reshape(1, A)
    wr = rpn_reg_w.reshape(4 * A, C).T
    br = rpn_reg_b.reshape(1, 4 * A)

    objs, dels, ancs, xs_nhwc = [], [], [], []
    for l, f in enumerate(feats):
        _, _, Hf, Wf = f.shape
        x = f.transpose(0, 2, 3, 1)
        xs_nhwc.append(x)
        o, d = _rpn_head_level(x, wk, bk, wc, bc, wr, br, H=Hf, W=Wf)
        objs.append(o.reshape(B, -1))
        dels.append(d.reshape(B, -1, 4))
        ancs.append(jnp.asarray(_level_anchors_np(Hf, Wf, STRIDES[l])))

    proposals, prop_scores = _rpn_proposals(objs, dels, ancs)
    boxes, scores = _roi_heads(
        xs_nhwc[:4], proposals, prop_scores,
        fc1_w, fc1_b, fc2_w, fc2_b, cls_w, cls_b, reg_w, reg_b)
    return boxes, scores


# speedup vs baseline: 4.0243x; 1.0447x over previous
import functools

import numpy as np
import jax
import jax.numpy as jnp
from jax import lax
from jax.experimental import pallas as pl
from jax.experimental.pallas import tpu as pltpu

H_IMG, W_IMG = 832.0, 640.0
STRIDES = (4, 8, 16, 32, 64)
SIZES = (64.0, 128.0, 256.0, 512.0, 1024.0)
RATIOS = (0.04, 0.05, 0.08, 0.15)
A = len(SIZES) * len(RATIOS)
PRE_NMS = 100
POST_NMS = 80
RPN_NMS_TH = 0.7
BOX_NMS_TH = 0.5
SCORE_TH = 0.05
POOL = 7
SAMPLING = 2
CANONICAL = 224.0
BBOX_W = (10.0, 10.0, 5.0, 5.0)
LOG_CLAMP = float(np.log(1000.0 / 16.0))

C = 256


def _base_anchors_np():
    scales = np.array(SIZES, np.float32)
    ratios = np.array(RATIOS, np.float32)
    h_r = np.sqrt(ratios)
    w_r = 1.0 / h_r
    ws = (w_r[:, None] * scales[None, :]).reshape(-1)
    hs = (h_r[:, None] * scales[None, :]).reshape(-1)
    return np.round(np.stack([-ws, -hs, ws, hs], axis=1) / 2.0)


def _level_anchors_np(hf, wf, stride):
    base = _base_anchors_np()
    sx = (np.arange(wf) * stride).astype(np.float32)
    sy = (np.arange(hf) * stride).astype(np.float32)
    shy, shx = np.meshgrid(sy, sx, indexing='ij')
    shifts = np.stack([shx, shy, shx, shy], axis=-1).reshape(-1, 1, 4)
    return (shifts + base[None]).reshape(-1, 4).astype(np.float32)


def _rpn_head_kernel(xm, xc, xp, wk, bk, wc, bc, wr, br, o_out, d_out, scr,
                     *, TH, W):
    i = pl.program_id(1)
    nH = pl.num_programs(1)
    zrow = jnp.zeros((1, W, C), jnp.float32)
    top = jnp.where(i > 0, xm[0, TH - 1:TH], zrow)
    bot = jnp.where(i < nH - 1, xp[0, 0:1], zrow)
    zcol = jnp.zeros((TH + 2, 1, C), jnp.float32)
    scr[:, 0:1, :] = zcol
    scr[:, W + 1:W + 2, :] = zcol
    scr[0:1, 1:W + 1, :] = top
    scr[1:TH + 1, 1:W + 1, :] = xc[0]
    scr[TH + 1:TH + 2, 1:W + 1, :] = bot

    acc = jnp.zeros((TH * W, C), jnp.float32)
    for dy in range(3):
        for dx in range(3):
            xs = scr[dy:dy + TH, dx:dx + W, :].reshape(TH * W, C)
            acc = acc + jnp.dot(xs, wk[dy, dx],
                                preferred_element_type=jnp.float32)
    t = jnp.maximum(acc + bk[0], 0.0)
    o = jnp.dot(t, wc[...], preferred_element_type=jnp.float32) + bc[0]
    d = jnp.dot(t, wr[...], preferred_element_type=jnp.float32) + br[0]
    o_out[0] = o.reshape(TH, W, A)
    d_out[0] = d.reshape(TH, W, 4 * A)


_LEVEL_TH = {208: 8, 104: 8, 52: 13, 26: 26, 13: 13}


@functools.partial(jax.jit, static_argnames=("H", "W"))
def _rpn_head_level(x, wk, bk, wc, bc, wr, br, *, H, W):
    B = x.shape[0]
    TH = _LEVEL_TH[H]
    nH = H // TH
    kfn = functools.partial(_rpn_head_kernel, TH=TH, W=W)
    o, d = pl.pallas_call(
        kfn,
        grid=(B, nH),
        in_specs=[
            pl.BlockSpec((1, TH, W, C), lambda b, i: (b, jnp.maximum(i - 1, 0), 0, 0)),
            pl.BlockSpec((1, TH, W, C), lambda b, i: (b, i, 0, 0)),
            pl.BlockSpec((1, TH, W, C), lambda b, i: (b, jnp.minimum(i + 1, nH - 1), 0, 0)),
            pl.BlockSpec((3, 3, C, C), lambda b, i: (0, 0, 0, 0)),
            pl.BlockSpec((1, C), lambda b, i: (0, 0)),
            pl.BlockSpec((C, A), lambda b, i: (0, 0)),
            pl.BlockSpec((1, A), lambda b, i: (0, 0)),
            pl.BlockSpec((C, 4 * A), lambda b, i: (0, 0)),
            pl.BlockSpec((1, 4 * A), lambda b, i: (0, 0)),
        ],
        out_specs=[
            pl.BlockSpec((1, TH, W, A), lambda b, i: (b, i, 0, 0)),
            pl.BlockSpec((1, TH, W, 4 * A), lambda b, i: (b, i, 0, 0)),
        ],
        out_shape=[
            jax.ShapeDtypeStruct((B, H, W, A), jnp.float32),
            jax.ShapeDtypeStruct((B, H, W, 4 * A), jnp.float32),
        ],
        scratch_shapes=[pltpu.VMEM((TH + 2, W + 2, C), jnp.float32)],
        compiler_params=pltpu.CompilerParams(
            dimension_semantics=("parallel", "arbitrary")),
    )(x, x, x, wk, bk, wc, bc, wr, br)
    return o, d


def _decode(deltas, boxes, weights):
    wx, wy, ww, wh = weights
    widths = boxes[:, 2] - boxes[:, 0]
    heights = boxes[:, 3] - boxes[:, 1]
    cx = boxes[:, 0] + 0.5 * widths
    cy = boxes[:, 1] + 0.5 * heights
    dx = deltas[:, 0] / wx
    dy = deltas[:, 1] / wy
    dw = jnp.minimum(deltas[:, 2] / ww, LOG_CLAMP)
    dh = jnp.minimum(deltas[:, 3] / wh, LOG_CLAMP)
    pcx = dx * widths + cx
    pcy = dy * heights + cy
    pw = jnp.exp(dw) * widths
    ph = jnp.exp(dh) * heights
    return jnp.stack([pcx - 0.5 * pw, pcy - 0.5 * ph,
                      pcx + 0.5 * pw, pcy + 0.5 * ph], axis=1)


def _clip_boxes(boxes):
    x = jnp.clip(boxes[:, 0::2], 0.0, W_IMG)
    y = jnp.clip(boxes[:, 1::2], 0.0, H_IMG)
    return jnp.stack([x[:, 0], y[:, 0], x[:, 1], y[:, 1]], axis=1)


def _nms_fix_kernel(b_ref, bt_ref, keep_ref, mref, *, N, thresh):
    x1c = jnp.broadcast_to(b_ref[0, :, 0:1], (N, N))
    y1c = jnp.broadcast_to(b_ref[0, :, 1:2], (N, N))
    x2c = jnp.broadcast_to(b_ref[0, :, 2:3], (N, N))
    y2c = jnp.broadcast_to(b_ref[0, :, 3:4], (N, N))
    x1r = jnp.broadcast_to(bt_ref[0, 0:1, :], (N, N))
    y1r = jnp.broadcast_to(bt_ref[0, 1:2, :], (N, N))
    x2r = jnp.broadcast_to(bt_ref[0, 2:3, :], (N, N))
    y2r = jnp.broadcast_to(bt_ref[0, 3:4, :], (N, N))
    areac = (x2c - x1c) * (y2c - y1c)
    arear = (x2r - x1r) * (y2r - y1r)
    inter = (jnp.clip(jnp.minimum(x2c, x2r) - jnp.maximum(x1c, x1r), 0.0)
             * jnp.clip(jnp.minimum(y2c, y2r) - jnp.maximum(y1c, y1r), 0.0))
    iou = inter / (areac + arear - inter + 1e-9)
    ii = lax.broadcasted_iota(jnp.int32, (N, N), 0)
    jj = lax.broadcasted_iota(jnp.int32, (N, N), 1)
    mref[...] = jnp.where((iou > thresh) & (jj > ii), 1.0, 0.0)

    def cond(c):
        _, ch, t = c
        return ch & (t < N + 2)

    def body(c):
        k, _, t = c
        sup = jnp.dot(k, mref[...], preferred_element_type=jnp.float32)
        nk = jnp.where(sup > 0.0, 0.0, 1.0)
        ch = jnp.sum(jnp.abs(nk - k)) > 0.0
        return nk, ch, t + 1

    k0 = jnp.ones((1, N), jnp.float32)
    kf, _, _ = lax.while_loop(cond, body, (k0, jnp.bool_(True), jnp.int32(0)))
    keep_ref[0] = kf


def _nms_pallas(boxes_sorted, thresh):
    B, N, _ = boxes_sorted.shape
    kfn = functools.partial(_nms_fix_kernel, N=N, thresh=thresh)
    keep = pl.pallas_call(
        kfn,
        grid=(B,),
        in_specs=[
            pl.BlockSpec((1, N, 4), lambda b: (b, 0, 0)),
            pl.BlockSpec((1, 4, N), lambda b: (b, 0, 0)),
        ],
        out_specs=pl.BlockSpec((1, 1, N), lambda b: (b, 0, 0)),
        out_shape=jax.ShapeDtypeStruct((B, 1, N), jnp.float32),
        scratch_shapes=[pltpu.VMEM((N, N), jnp.float32)],
        compiler_params=pltpu.CompilerParams(
            dimension_semantics=("parallel",)),
    )(boxes_sorted, boxes_sorted.transpose(0, 2, 1))
    return keep[:, 0, :]


def _pool_one(scr, out_ref, r, x1, y1, x2, y2, H, W):
    rw = jnp.maximum(x2 - x1, 1.0)
    rh = jnp.maximum(y2 - y1, 1.0)
    stepx = rw / float(POOL) / float(SAMPLING)
    stepy = rh / float(POOL) / float(SAMPLING)
    pr = lax.broadcasted_iota(jnp.int32, (POOL, W), 0).astype(jnp.float32)
    xi = lax.broadcasted_iota(jnp.int32, (POOL, W), 1).astype(jnp.float32)
    g = jnp.zeros((POOL, W), jnp.float32)
    for s in (0, 1):
        xs = x1 + (2.0 * pr + (s + 0.5)) * stepx
        xv = jnp.where((xs > -1.0) & (xs < float(W)), 1.0, 0.0)
        xc = jnp.clip(xs, 0.0, float(W - 1))
        x0 = jnp.floor(xc)
        x1i = jnp.minimum(x0 + 1.0, float(W - 1))
        lx = xc - x0
        g = (g + jnp.where(xi == x0, (1.0 - lx) * xv, 0.0)
             + jnp.where(xi == x1i, lx * xv, 0.0))
    rows = []
    for pp in range(POOL):
        racc = jnp.zeros((W, C), jnp.float32)
        for s in (0, 1):
            yy = y1 + (2.0 * pp + (s + 0.5)) * stepy
            yv = jnp.where((yy > -1.0) & (yy < float(H)), 1.0, 0.0)
            yc = jnp.clip(yy, 0.0, float(H - 1))
            y0f = jnp.floor(yc)
            ly = yc - y0f
            y0 = y0f.astype(jnp.int32)
            y1i = jnp.minimum(y0 + 1, H - 1)
            racc = racc + (yv * (1.0 - ly)) * scr[y0] + (yv * ly) * scr[y1i]
        rows.append(jnp.dot(g, racc, preferred_element_type=jnp.float32) * 0.25)
    out_ref[0, r] = jnp.concatenate(rows, axis=0)


def _roi_pool_kernel(f0, f1, f2, f3, rl, out, s0, s1, s2, s3, sems):
    b = pl.program_id(0)
    frefs = (f0, f1, f2, f3)
    scrs = (s0, s1, s2, s3)
    for l in range(4):
        pltpu.make_async_copy(frefs[l].at[b], scrs[l], sems.at[l]).start()
    for l in range(4):
        pltpu.make_async_copy(frefs[l].at[b], scrs[l], sems.at[l]).wait()

    def body(r, carry):
        lv = rl[0, r, 4]
        for l in range(4):
            @pl.when(lv == float(l))
            def _(l=l):
                sc = 1.0 / float(STRIDES[l])
                Hl, Wl = scrs[l].shape[0], scrs[l].shape[1]
                _pool_one(scrs[l], out, r,
                          rl[0, r, 0] * sc, rl[0, r, 1] * sc,
                          rl[0, r, 2] * sc, rl[0, r, 3] * sc, Hl, Wl)
        return carry

    lax.fori_loop(0, POST_NMS, body, jnp.int32(0))


def _roi_pool_pallas(xs_nhwc, rois):
    B = rois.shape[0]
    area = (jnp.clip(rois[:, :, 2] - rois[:, :, 0], 0.0)
            * jnp.clip(rois[:, :, 3] - rois[:, :, 1], 0.0))
    lvl = jnp.floor(4.0 + jnp.log2(jnp.sqrt(area) / CANONICAL + 1e-6))
    lvl = jnp.clip(lvl, 2.0, 5.0) - 2.0
    rl = jnp.concatenate(
        [rois, lvl[:, :, None], jnp.zeros((B, POST_NMS, 3), jnp.float32)],
        axis=-1)
    pooled = pl.pallas_call(
        _roi_pool_kernel,
        grid=(B,),
        in_specs=[
            pl.BlockSpec(memory_space=pl.ANY),
            pl.BlockSpec(memory_space=pl.ANY),
            pl.BlockSpec(memory_space=pl.ANY),
            pl.BlockSpec(memory_space=pl.ANY),
            pl.BlockSpec((1, POST_NMS, 8), lambda b: (b, 0, 0),
                         memory_space=pltpu.SMEM),
        ],
        out_specs=pl.BlockSpec((1, POST_NMS, POOL * POOL, C),
                               lambda b: (b, 0, 0, 0)),
        out_shape=jax.ShapeDtypeStruct((B, POST_NMS, POOL * POOL, C),
                                       jnp.float32),
        scratch_shapes=[
            pltpu.VMEM(xs_nhwc[0].shape[1:], jnp.float32),
            pltpu.VMEM(xs_nhwc[1].shape[1:], jnp.float32),
            pltpu.VMEM(xs_nhwc[2].shape[1:], jnp.float32),
            pltpu.VMEM(xs_nhwc[3].shape[1:], jnp.float32),
            pltpu.SemaphoreType.DMA((4,)),
        ],
        compiler_params=pltpu.CompilerParams(
            dimension_semantics=("parallel",)),
    )(*xs_nhwc, rl)
    return pooled


def _take2(x, idx):
    if x.ndim == 3:
        return jnp.take_along_axis(x, idx[:, :, None], axis=1)
    return jnp.take_along_axis(x, idx, axis=1)


def _rpn_proposals(objs, dels, ancs):
    B = objs[0].shape[0]
    boxes_l, scores_l, lvl_l = [], [], []
    for l in range(len(objs)):
        o, d, anc = objs[l], dels[l], ancs[l]
        k = min(PRE_NMS, o.shape[1])
        top_s, top_i = lax.approx_max_k(o, k, recall_target=1.0)
        dk = _take2(d, top_i).reshape(B * k, 4)
        ak = anc[top_i].reshape(B * k, 4)
        bx = _clip_boxes(_decode(dk, ak, (1.0, 1.0, 1.0, 1.0))).reshape(B, k, 4)
        boxes_l.append(bx)
        scores_l.append(jax.nn.sigmoid(top_s))
        lvl_l.append(jnp.full((B, k), float(l), jnp.float32))
    boxes = jnp.concatenate(boxes_l, axis=1)
    scores = jnp.concatenate(scores_l, axis=1)
    lvls = jnp.concatenate(lvl_l, axis=1)
    valid = ((boxes[:, :, 2] - boxes[:, :, 0] >= 1e-3) &
             (boxes[:, :, 3] - boxes[:, :, 1] >= 1e-3))
    scores = jnp.where(valid, scores, -jnp.inf)
    off = lvls[:, :, None] * (max(H_IMG, W_IMG) + 2.0)
    N = boxes.shape[1]
    NP = 512
    pb = jnp.zeros((B, NP - N, 4), jnp.float32)
    boxes_p = jnp.concatenate([boxes, pb], axis=1)
    off_p = jnp.concatenate([boxes + off, pb], axis=1)
    scores_p = jnp.concatenate(
        [scores, jnp.full((B, NP - N), -jnp.inf, jnp.float32)], axis=1)
    order = jnp.argsort(-scores_p, axis=1)
    keep_s = _nms_pallas(_take2(off_p, order), RPN_NMS_TH)
    final_s = jnp.where(keep_s > 0, _take2(scores_p, order), -jnp.inf)
    top_s, top_i = lax.top_k(final_s, POST_NMS)
    proposals = _take2(_take2(boxes_p, order), top_i)
    return proposals, top_s


def _roi_heads(xs_nhwc, proposals, prop_scores,
               fc1_w, fc1_b, fc2_w, fc2_b, cls_w, cls_b, reg_w, reg_b):
    B = proposals.shape[0]
    pooled = _roi_pool_pallas(xs_nhwc, proposals)
    x = jnp.einsum('bnpc,cpj->bnj', pooled,
                   fc1_w.reshape(C, POOL * POOL, fc1_w.shape[1]),
                   preferred_element_type=jnp.float32)
    x = jax.nn.relu(x + fc1_b)
    x = jax.nn.relu(x @ fc2_w + fc2_b)
    logits = x @ cls_w + cls_b
    deltas = x @ reg_w + reg_b
    scores = jax.nn.softmax(logits, axis=-1)[:, :, 1]
    boxes = _clip_boxes(
        _decode(deltas[:, :, 4:8].reshape(B * POST_NMS, 4),
                proposals.reshape(B * POST_NMS, 4), BBOX_W)
    ).reshape(B, POST_NMS, 4)
    valid = jnp.isfinite(prop_scores)
    valid &= (boxes[:, :, 2] - boxes[:, :, 0] >= 1e-2) & (boxes[:, :, 3] - boxes[:, :, 1] >= 1e-2)
    valid &= scores > SCORE_TH
    s = jnp.where(valid, scores, -jnp.inf)
    N = boxes.shape[1]
    NP = 128
    boxes_p = jnp.concatenate(
        [boxes, jnp.zeros((B, NP - N, 4), jnp.float32)], axis=1)
    s_p = jnp.concatenate(
        [s, jnp.full((B, NP - N), -jnp.inf, jnp.float32)], axis=1)
    order = jnp.argsort(-s_p, axis=1)
    keep_sorted = _nms_pallas(_take2(boxes_p, order), BOX_NMS_TH)
    keep = jnp.zeros((B, NP), bool).at[
        jnp.arange(B)[:, None], order].set(keep_sorted > 0)[:, :N]
    keep &= valid
    return jnp.where(keep[:, :, None], boxes, 0.0), jnp.where(keep, scores, 0.0)


def kernel(feats0, feats1, feats2, feats3, feats4,
           rpn_conv_w, rpn_conv_b, rpn_cls_w, rpn_cls_b, rpn_reg_w, rpn_reg_b,
           fc1_w, fc1_b, fc2_w, fc2_b, cls_w, cls_b, reg_w, reg_b):
    feats = (feats0, feats1, feats2, feats3, feats4)
    B = feats0.shape[0]

    wk = rpn_conv_w.transpose(2, 3, 1, 0)
    bk = rpn_conv_b.reshape(1, C)
    wc = rpn_cls_w.reshape(A, C).T
    bc = rpn_cls_b.reshape(1, A)
    wr = rpn_reg_w.reshape(4 * A, C).T
    br = rpn_reg_b.reshape(1, 4 * A)

    objs, dels, ancs, xs_nhwc = [], [], [], []
    for l, f in enumerate(feats):
        _, _, Hf, Wf = f.shape
        x = f.transpose(0, 2, 3, 1)
        xs_nhwc.append(x)
        o, d = _rpn_head_level(x, wk, bk, wc, bc, wr, br, H=Hf, W=Wf)
        objs.append(o.reshape(B, -1))
        dels.append(d.reshape(B, -1, 4))
        ancs.append(jnp.asarray(_level_anchors_np(Hf, Wf, STRIDES[l])))

    proposals, prop_scores = _rpn_proposals(objs, dels, ancs)
    boxes, scores = _roi_heads(
        xs_nhwc[:4], proposals, prop_scores,
        fc1_w, fc1_b, fc2_w, fc2_b, cls_w, cls_b, reg_w, reg_b)
    return boxes, scores
